# baseline (speedup 1.0000x reference)
import jax
import jax.numpy as jnp
from jax import lax
from jax.experimental import pallas as pl
from jax.experimental.pallas import tpu as pltpu
from jax.experimental.pallas import tpu_sc as plsc

_N_NODES = 10000
_N_EDGES = 320000
_D = 128

_NC = 2
_NS = 16
_NW = _NC * _NS
_CHUNK = 128
_CHUNKS_PER_W = 80
_E_PAD = _NW * _CHUNKS_PER_W * _CHUNK
_DUMP_ROW = _N_NODES
_ACC_ROWS = 10112
_ROWS_PER_SUB = _ACC_ROWS // _NS
_GCHUNK = 16
_GROUPS = _CHUNKS_PER_W // _GCHUNK
_MM_BLOCK = 2000


def _mm_body(x_ref, w_ref, o_ref):
    o_ref[...] = jnp.dot(x_ref[...], w_ref[...],
                         preferred_element_type=jnp.float32)


def _add_body(a_ref, b_ref, o_ref):
    o_ref[...] = a_ref[...] + b_ref[...]


def _sc_body(sup_hbm, src_hbm, dst_hbm, zeros_hbm, out_hbm,
             src_blk, dst_blk, rows0, rows1, acc,
             sem0a, sem0b, sem1a, sem1b, sems0, sems1, semi):
    cid = lax.axis_index("c")
    sid = lax.axis_index("s")
    wid = sid * _NC + cid

    pltpu.sync_copy(zeros_hbm.at[pl.ds(sid * _ROWS_PER_SUB, _ROWS_PER_SUB)],
                    acc.at[pl.ds(sid * _ROWS_PER_SUB, _ROWS_PER_SUB)])
    plsc.subcore_barrier()

    def gather(g, c, rows, sema, semb):
        del semb
        return pltpu.make_async_copy(sup_hbm.at[src_blk.at[g % 2].at[c]],
                                     rows, sema)

    def g_start(g, c, rows, sema, semb):
        gather(g, c, rows, sema, semb).start()

    def g_wait(g, c, rows, sema, semb):
        gather(g, c, rows, sema, semb).wait()

    def scat(g, c, rows, sem):
        d = pltpu.make_async_copy(rows, acc.at[dst_blk.at[g % 2].at[c]], sem)
        d.start(add=True)
        return d

    def idx_copies(g):
        p = g % 2
        return (
            pltpu.make_async_copy(
                src_hbm.at[wid].at[pl.ds(g * _GCHUNK, _GCHUNK)],
                src_blk.at[p], semi),
            pltpu.make_async_copy(
                dst_hbm.at[wid].at[pl.ds(g * _GCHUNK, _GCHUNK)],
                dst_blk.at[p], semi),
        )

    ia, ib = idx_copies(0)
    ia.start()
    ib.start()
    ia.wait()
    ib.wait()
    g_start(0, 0, rows0, sem0a, sem0b)
    g_start(0, 1, rows1, sem1a, sem1b)
    if _GROUPS > 1:
        ia, ib = idx_copies(1)
        ia.start()
        ib.start()

    for g in range(_GROUPS):
        for i in range(_GCHUNK // 2):
            c0, c1 = 2 * i, 2 * i + 1
            g_wait(g, c0, rows0, sem0a, sem0b)
            s0 = scat(g, c0, rows0, sems0)
            g_wait(g, c1, rows1, sem1a, sem1b)
            s1 = scat(g, c1, rows1, sems1)
            last_pair = (i == _GCHUNK // 2 - 1)
            if not last_pair:
                s0.wait()
                g_start(g, c0 + 2, rows0, sem0a, sem0b)
                s1.wait()
                g_start(g, c1 + 2, rows1, sem1a, sem1b)
            elif g + 1 < _GROUPS:
                ia, ib = idx_copies(g + 1)
                ia.wait()
                ib.wait()
                s0.wait()
                g_start(g + 1, 0, rows0, sem0a, sem0b)
                s1.wait()
                g_start(g + 1, 1, rows1, sem1a, sem1b)
                if g + 2 < _GROUPS:
                    ia, ib = idx_copies(g + 2)
                    ia.start()
                    ib.start()
            else:
                s0.wait()
                s1.wait()

    plsc.subcore_barrier()
    pltpu.sync_copy(
        acc.at[pl.ds(sid * _ROWS_PER_SUB, _ROWS_PER_SUB)],
        out_hbm.at[cid].at[pl.ds(sid * _ROWS_PER_SUB, _ROWS_PER_SUB)])


@jax.jit
def kernel(features, edge_index, W):
    support = pl.pallas_call(
        _mm_body,
        grid=(_N_NODES // _MM_BLOCK,),
        in_specs=[
            pl.BlockSpec((_MM_BLOCK, _D), lambda i: (i, 0)),
            pl.BlockSpec((_D, _D), lambda i: (0, 0)),
        ],
        out_specs=pl.BlockSpec((_MM_BLOCK, _D), lambda i: (i, 0)),
        out_shape=jax.ShapeDtypeStruct((_N_NODES, _D), jnp.float32),
    )(features, W)

    real_per_w = _N_EDGES // _NW
    pad_per_w = _CHUNKS_PER_W * _CHUNK - real_per_w
    n_dump = _ACC_ROWS - _N_NODES
    src_r = edge_index[0].reshape(_NW, real_per_w)
    dst_r = edge_index[1].reshape(_NW, real_per_w)
    pad_dst = _N_NODES + (jnp.arange(pad_per_w, dtype=jnp.int32) % n_dump)
    src_p = jnp.concatenate(
        [src_r, jnp.zeros((_NW, pad_per_w), jnp.int32)], axis=1)
    dst_p = jnp.concatenate(
        [dst_r, jnp.broadcast_to(pad_dst, (_NW, pad_per_w))], axis=1)
    src_p = src_p.reshape(_NW, _CHUNKS_PER_W, _CHUNK)
    dst_p = dst_p.reshape(_NW, _CHUNKS_PER_W, _CHUNK)
    zeros = jnp.zeros((_ACC_ROWS, _D), jnp.float32)

    sc_call = pl.kernel(
        _sc_body,
        out_type=jax.ShapeDtypeStruct((_NC, _ACC_ROWS, _D), jnp.float32),
        mesh=plsc.VectorSubcoreMesh(core_axis_name="c", subcore_axis_name="s"),
        scratch_types=[
            pltpu.VMEM((2, _GCHUNK, _CHUNK), jnp.int32),
            pltpu.VMEM((2, _GCHUNK, _CHUNK), jnp.int32),
            pltpu.VMEM((_CHUNK, _D), jnp.float32),
            pltpu.VMEM((_CHUNK, _D), jnp.float32),
            pltpu.VMEM_SHARED((_ACC_ROWS, _D), jnp.float32),
            pltpu.SemaphoreType.DMA,
            pltpu.SemaphoreType.DMA,
            pltpu.SemaphoreType.DMA,
            pltpu.SemaphoreType.DMA,
            pltpu.SemaphoreType.DMA,
            pltpu.SemaphoreType.DMA,
            pltpu.SemaphoreType.DMA,
        ],
    )
    partials = sc_call(support, src_p, dst_p, zeros)

    out = pl.pallas_call(
        _add_body,
        grid=(_N_NODES // _MM_BLOCK,),
        in_specs=[
            pl.BlockSpec((_MM_BLOCK, _D), lambda i: (i, 0)),
            pl.BlockSpec((_MM_BLOCK, _D), lambda i: (i, 0)),
        ],
        out_specs=pl.BlockSpec((_MM_BLOCK, _D), lambda i: (i, 0)),
        out_shape=jax.ShapeDtypeStruct((_N_NODES, _D), jnp.float32),
    )(partials[0], partials[1])
    return out


# speedup vs baseline: 3.0334x; 3.0334x over previous
import jax
import jax.numpy as jnp
from jax import lax
from jax.experimental import pallas as pl
from jax.experimental.pallas import tpu as pltpu
from jax.experimental.pallas import tpu_sc as plsc

_N_NODES = 10000
_N_EDGES = 320000
_D = 128

_NC = 2
_NS = 16
_NW = _NC * _NS
_CHUNK = 80
_CHUNKS_PER_W = 125
_NSLOT = 4
_ACC_ROWS = 10112
_ROWS_PER_SUB = _ACC_ROWS // _NS
_GCHUNK = 16
_GROUPS = 8
_CHUNKS_PAD = _GROUPS * _GCHUNK
_MM_BLOCK = 2000


def _mm_body(x_ref, w_ref, o_ref):
    o_ref[...] = jnp.dot(x_ref[...], w_ref[...],
                         preferred_element_type=jnp.float32)


def _add_body(a_ref, b_ref, o_ref):
    o_ref[...] = a_ref[...] + b_ref[...]


def _sc_body(sup_hbm, src_hbm, dst_hbm, zeros_hbm, out_hbm,
             src_blk, dst_blk, rows0, rows1, rows2, rows3, acc,
             semg0, semg1, semg2, semg3, sems0, sems1, sems2, sems3, semi):
    cid = lax.axis_index("c")
    sid = lax.axis_index("s")
    wid = sid * _NC + cid

    rows = [rows0, rows1, rows2, rows3]
    gsem = [semg0, semg1, semg2, semg3]
    ssem = [sems0, sems1, sems2, sems3]

    def gather(c):
        g, k = divmod(c, _GCHUNK)
        r = c % _NSLOT
        return pltpu.make_async_copy(sup_hbm.at[src_blk.at[g % 2].at[k]],
                                     rows[r], gsem[r])

    def scat(c):
        g, k = divmod(c, _GCHUNK)
        r = c % _NSLOT
        d = pltpu.make_async_copy(rows[r], acc.at[dst_blk.at[g % 2].at[k]],
                                  ssem[r])
        d.start(add=True)
        return d

    def s_wait(c):
        scatter_done = pltpu.make_async_copy(
            rows[c % _NSLOT], acc.at[dst_blk.at[(c // _GCHUNK) % 2].at[0]],
            ssem[c % _NSLOT])
        scatter_done.wait()

    def idx_copies(g):
        p = g % 2
        return (
            pltpu.make_async_copy(src_hbm.at[wid].at[g], src_blk.at[p], semi),
            pltpu.make_async_copy(dst_hbm.at[wid].at[g], dst_blk.at[p], semi),
        )

    ia, ib = idx_copies(0)
    ia.start()
    ib.start()
    ia.wait()
    ib.wait()
    gather(0).start()
    gather(1).start()
    pltpu.sync_copy(zeros_hbm.at[pl.ds(sid * _ROWS_PER_SUB, _ROWS_PER_SUB)],
                    acc.at[pl.ds(sid * _ROWS_PER_SUB, _ROWS_PER_SUB)])
    plsc.subcore_barrier()

    for c in range(_CHUNKS_PER_W):
        if c >= 2:
            s_wait(c - 2)
        if c + 2 < _CHUNKS_PER_W:
            if (c + 2) % _GCHUNK == 0:
                ia, ib = idx_copies((c + 2) // _GCHUNK)
                ia.wait()
                ib.wait()
            gather(c + 2).start()
        if c % _GCHUNK == 1 and c // _GCHUNK + 1 < _GROUPS:
            ia, ib = idx_copies(c // _GCHUNK + 1)
            ia.start()
            ib.start()
        gather(c).wait()
        scat(c)

    s_wait(_CHUNKS_PER_W - 2)
    s_wait(_CHUNKS_PER_W - 1)

    plsc.subcore_barrier()
    pltpu.sync_copy(
        acc.at[pl.ds(sid * _ROWS_PER_SUB, _ROWS_PER_SUB)],
        out_hbm.at[cid].at[pl.ds(sid * _ROWS_PER_SUB, _ROWS_PER_SUB)])


@jax.jit
def kernel(features, edge_index, W):
    support = pl.pallas_call(
        _mm_body,
        grid=(_N_NODES // _MM_BLOCK,),
        in_specs=[
            pl.BlockSpec((_MM_BLOCK, _D), lambda i: (i, 0)),
            pl.BlockSpec((_D, _D), lambda i: (0, 0)),
        ],
        out_specs=pl.BlockSpec((_MM_BLOCK, _D), lambda i: (i, 0)),
        out_shape=jax.ShapeDtypeStruct((_N_NODES, _D), jnp.float32),
    )(features, W)

    pad_c = jnp.zeros((_NW, _CHUNKS_PAD - _CHUNKS_PER_W, _CHUNK), jnp.int32)
    src_p = jnp.concatenate(
        [edge_index[0].reshape(_NW, _CHUNKS_PER_W, _CHUNK), pad_c], axis=1
    ).reshape(_NW, _GROUPS, _GCHUNK, _CHUNK)
    dst_p = jnp.concatenate(
        [edge_index[1].reshape(_NW, _CHUNKS_PER_W, _CHUNK), pad_c], axis=1
    ).reshape(_NW, _GROUPS, _GCHUNK, _CHUNK)
    zeros = jnp.zeros((_ACC_ROWS, _D), jnp.float32)

    sc_call = pl.kernel(
        _sc_body,
        out_type=jax.ShapeDtypeStruct((_NC, _ACC_ROWS, _D), jnp.float32),
        mesh=plsc.VectorSubcoreMesh(core_axis_name="c", subcore_axis_name="s"),
        scratch_types=[
            pltpu.VMEM((2, _GCHUNK, _CHUNK), jnp.int32),
            pltpu.VMEM((2, _GCHUNK, _CHUNK), jnp.int32),
            pltpu.VMEM((_CHUNK, _D), jnp.float32),
            pltpu.VMEM((_CHUNK, _D), jnp.float32),
            pltpu.VMEM((_CHUNK, _D), jnp.float32),
            pltpu.VMEM((_CHUNK, _D), jnp.float32),
            pltpu.VMEM_SHARED((_ACC_ROWS, _D), jnp.float32),
            pltpu.SemaphoreType.DMA,
            pltpu.SemaphoreType.DMA,
            pltpu.SemaphoreType.DMA,
            pltpu.SemaphoreType.DMA,
            pltpu.SemaphoreType.DMA,
            pltpu.SemaphoreType.DMA,
            pltpu.SemaphoreType.DMA,
            pltpu.SemaphoreType.DMA,
            pltpu.SemaphoreType.DMA,
        ],
    )
    partials = sc_call(support, src_p, dst_p, zeros)

    out = pl.pallas_call(
        _add_body,
        grid=(_N_NODES // _MM_BLOCK,),
        in_specs=[
            pl.BlockSpec((_MM_BLOCK, _D), lambda i: (i, 0)),
            pl.BlockSpec((_MM_BLOCK, _D), lambda i: (i, 0)),
        ],
        out_specs=pl.BlockSpec((_MM_BLOCK, _D), lambda i: (i, 0)),
        out_shape=jax.ShapeDtypeStruct((_N_NODES, _D), jnp.float32),
    )(partials[0], partials[1])
    return out


# speedup vs baseline: 3.1649x; 1.0434x over previous
import jax
import jax.numpy as jnp
from jax import lax
from jax.experimental import pallas as pl
from jax.experimental.pallas import tpu as pltpu
from jax.experimental.pallas import tpu_sc as plsc

_N_NODES = 10000
_N_EDGES = 320000
_D = 128

_NC = 2
_NS = 16
_NW = _NC * _NS
_CHUNK = 80
_CHUNKS_PER_W = 125
_NSLOT = 4
_ACC_ROWS = 10112
_ROWS_PER_SUB = _ACC_ROWS // _NS
_GCHUNK = 16
_GROUPS = 8
_CHUNKS_PAD = _GROUPS * _GCHUNK
_MM_BLOCK = 2000


def _addmm_body(a_ref, b_ref, w_ref, o_ref):
    o_ref[...] = jnp.dot(a_ref[...] + b_ref[...], w_ref[...],
                         preferred_element_type=jnp.float32)


def _sc_body(sup_hbm, src_hbm, dst_hbm, zeros_hbm, out_hbm,
             src_blk, dst_blk, rows0, rows1, rows2, rows3, acc,
             semg0, semg1, semg2, semg3, sems0, sems1, sems2, sems3, semi):
    cid = lax.axis_index("c")
    sid = lax.axis_index("s")
    wid = sid * _NC + cid

    rows = [rows0, rows1, rows2, rows3]
    gsem = [semg0, semg1, semg2, semg3]
    ssem = [sems0, sems1, sems2, sems3]

    def gather(c):
        g, k = divmod(c, _GCHUNK)
        r = c % _NSLOT
        return pltpu.make_async_copy(sup_hbm.at[src_blk.at[g % 2].at[k]],
                                     rows[r], gsem[r])

    def scat(c):
        g, k = divmod(c, _GCHUNK)
        r = c % _NSLOT
        d = pltpu.make_async_copy(rows[r], acc.at[dst_blk.at[g % 2].at[k]],
                                  ssem[r])
        d.start(add=True)
        return d

    def s_wait(c):
        scatter_done = pltpu.make_async_copy(
            rows[c % _NSLOT], acc.at[dst_blk.at[(c // _GCHUNK) % 2].at[0]],
            ssem[c % _NSLOT])
        scatter_done.wait()

    def idx_copies(g):
        p = g % 2
        return (
            pltpu.make_async_copy(src_hbm.at[wid].at[g], src_blk.at[p], semi),
            pltpu.make_async_copy(dst_hbm.at[wid].at[g], dst_blk.at[p], semi),
        )

    ia, ib = idx_copies(0)
    ia.start()
    ib.start()
    ia.wait()
    ib.wait()
    gather(0).start()
    gather(1).start()
    pltpu.sync_copy(zeros_hbm.at[pl.ds(sid * _ROWS_PER_SUB, _ROWS_PER_SUB)],
                    acc.at[pl.ds(sid * _ROWS_PER_SUB, _ROWS_PER_SUB)])
    plsc.subcore_barrier()

    for c in range(_CHUNKS_PER_W):
        if c >= 2:
            s_wait(c - 2)
        if c + 2 < _CHUNKS_PER_W:
            if (c + 2) % _GCHUNK == 0:
                ia, ib = idx_copies((c + 2) // _GCHUNK)
                ia.wait()
                ib.wait()
            gather(c + 2).start()
        if c % _GCHUNK == 1 and c // _GCHUNK + 1 < _GROUPS:
            ia, ib = idx_copies(c // _GCHUNK + 1)
            ia.start()
            ib.start()
        gather(c).wait()
        scat(c)

    s_wait(_CHUNKS_PER_W - 2)
    s_wait(_CHUNKS_PER_W - 1)

    plsc.subcore_barrier()
    pltpu.sync_copy(
        acc.at[pl.ds(sid * _ROWS_PER_SUB, _ROWS_PER_SUB)],
        out_hbm.at[cid].at[pl.ds(sid * _ROWS_PER_SUB, _ROWS_PER_SUB)])


@jax.jit
def kernel(features, edge_index, W):
    pad_c = jnp.zeros((_NW, _CHUNKS_PAD - _CHUNKS_PER_W, _CHUNK), jnp.int32)
    src_p = jnp.concatenate(
        [edge_index[0].reshape(_NW, _CHUNKS_PER_W, _CHUNK), pad_c], axis=1
    ).reshape(_NW, _GROUPS, _GCHUNK, _CHUNK)
    dst_p = jnp.concatenate(
        [edge_index[1].reshape(_NW, _CHUNKS_PER_W, _CHUNK), pad_c], axis=1
    ).reshape(_NW, _GROUPS, _GCHUNK, _CHUNK)
    zeros = jnp.zeros((_ACC_ROWS, _D), jnp.float32)

    sc_call = pl.kernel(
        _sc_body,
        out_type=jax.ShapeDtypeStruct((_NC, _ACC_ROWS, _D), jnp.float32),
        mesh=plsc.VectorSubcoreMesh(core_axis_name="c", subcore_axis_name="s"),
        scratch_types=[
            pltpu.VMEM((2, _GCHUNK, _CHUNK), jnp.int32),
            pltpu.VMEM((2, _GCHUNK, _CHUNK), jnp.int32),
            pltpu.VMEM((_CHUNK, _D), jnp.float32),
            pltpu.VMEM((_CHUNK, _D), jnp.float32),
            pltpu.VMEM((_CHUNK, _D), jnp.float32),
            pltpu.VMEM((_CHUNK, _D), jnp.float32),
            pltpu.VMEM_SHARED((_ACC_ROWS, _D), jnp.float32),
            pltpu.SemaphoreType.DMA,
            pltpu.SemaphoreType.DMA,
            pltpu.SemaphoreType.DMA,
            pltpu.SemaphoreType.DMA,
            pltpu.SemaphoreType.DMA,
            pltpu.SemaphoreType.DMA,
            pltpu.SemaphoreType.DMA,
            pltpu.SemaphoreType.DMA,
            pltpu.SemaphoreType.DMA,
        ],
    )
    partials = sc_call(features, src_p, dst_p, zeros)

    out = pl.pallas_call(
        _addmm_body,
        grid=(_N_NODES // _MM_BLOCK,),
        in_specs=[
            pl.BlockSpec((_MM_BLOCK, _D), lambda i: (i, 0)),
            pl.BlockSpec((_MM_BLOCK, _D), lambda i: (i, 0)),
            pl.BlockSpec((_D, _D), lambda i: (0, 0)),
        ],
        out_specs=pl.BlockSpec((_MM_BLOCK, _D), lambda i: (i, 0)),
        out_shape=jax.ShapeDtypeStruct((_N_NODES, _D), jnp.float32),
    )(partials[0], partials[1], W)
    return out


# speedup vs baseline: 3.1936x; 1.0091x over previous
import jax
import jax.numpy as jnp
from jax import lax
from jax.experimental import pallas as pl
from jax.experimental.pallas import tpu as pltpu
from jax.experimental.pallas import tpu_sc as plsc

_N_NODES = 10000
_N_EDGES = 320000
_D = 128

_NC = 2
_NS = 16
_NW = _NC * _NS
_CHUNK = 80
_CHUNKS_PER_W = 125
_NSLOT = 4
_ACC_ROWS = 10112
_ROWS_PER_SUB = _ACC_ROWS // _NS
_GCHUNK = 16
_GROUPS = 8
_MM_BLOCK = 2000


def _addmm_body(a_ref, b_ref, w_ref, o_ref):
    o_ref[...] = jnp.dot(a_ref[...] + b_ref[...], w_ref[...],
                         preferred_element_type=jnp.float32)


def _sc_body(sup_hbm, src_hbm, dst_hbm, zeros_hbm, out_hbm,
             src_blk, dst_blk, rows0, rows1, rows2, rows3, acc,
             semg0, semg1, semg2, semg3, sems0, sems1, sems2, sems3, semi):
    cid = lax.axis_index("c")
    sid = lax.axis_index("s")
    wid = sid * _NC + cid

    rows = [rows0, rows1, rows2, rows3]
    gsem = [semg0, semg1, semg2, semg3]
    ssem = [sems0, sems1, sems2, sems3]

    def gather(c):
        g, k = divmod(c, _GCHUNK)
        r = c % _NSLOT
        return pltpu.make_async_copy(sup_hbm.at[src_blk.at[g % 2].at[k]],
                                     rows[r], gsem[r])

    def scat(c):
        g, k = divmod(c, _GCHUNK)
        r = c % _NSLOT
        d = pltpu.make_async_copy(rows[r], acc.at[dst_blk.at[g % 2].at[k]],
                                  ssem[r])
        d.start(add=True)
        return d

    def s_wait(c):
        scatter_done = pltpu.make_async_copy(
            rows[c % _NSLOT], acc.at[dst_blk.at[(c // _GCHUNK) % 2].at[0]],
            ssem[c % _NSLOT])
        scatter_done.wait()

    def idx_copies(g):
        p = g % 2
        n = min(_GCHUNK, _CHUNKS_PER_W - g * _GCHUNK)
        return (
            pltpu.make_async_copy(
                src_hbm.at[wid].at[pl.ds(g * _GCHUNK, n)],
                src_blk.at[p].at[pl.ds(0, n)], semi),
            pltpu.make_async_copy(
                dst_hbm.at[wid].at[pl.ds(g * _GCHUNK, n)],
                dst_blk.at[p].at[pl.ds(0, n)], semi),
        )

    ia, ib = idx_copies(0)
    ia.start()
    ib.start()
    ia.wait()
    ib.wait()
    gather(0).start()
    gather(1).start()
    pltpu.sync_copy(zeros_hbm,
                    acc.at[pl.ds(sid * _ROWS_PER_SUB, _ROWS_PER_SUB)])
    plsc.subcore_barrier()

    for c in range(_CHUNKS_PER_W):
        if c >= 2:
            s_wait(c - 2)
        if c + 2 < _CHUNKS_PER_W:
            if (c + 2) % _GCHUNK == 0:
                ia, ib = idx_copies((c + 2) // _GCHUNK)
                ia.wait()
                ib.wait()
            gather(c + 2).start()
        if c % _GCHUNK == 1 and c // _GCHUNK + 1 < _GROUPS:
            ia, ib = idx_copies(c // _GCHUNK + 1)
            ia.start()
            ib.start()
        gather(c).wait()
        scat(c)

    s_wait(_CHUNKS_PER_W - 2)
    s_wait(_CHUNKS_PER_W - 1)

    plsc.subcore_barrier()
    pltpu.sync_copy(
        acc.at[pl.ds(sid * _ROWS_PER_SUB, _ROWS_PER_SUB)],
        out_hbm.at[cid].at[pl.ds(sid * _ROWS_PER_SUB, _ROWS_PER_SUB)])


@jax.jit
def kernel(features, edge_index, W):
    src_p = edge_index[0].reshape(_NW, _CHUNKS_PER_W, _CHUNK)
    dst_p = edge_index[1].reshape(_NW, _CHUNKS_PER_W, _CHUNK)
    zeros = jnp.zeros((_ROWS_PER_SUB, _D), jnp.float32)

    sc_call = pl.kernel(
        _sc_body,
        out_type=jax.ShapeDtypeStruct((_NC, _ACC_ROWS, _D), jnp.float32),
        mesh=plsc.VectorSubcoreMesh(core_axis_name="c", subcore_axis_name="s"),
        scratch_types=[
            pltpu.VMEM((2, _GCHUNK, _CHUNK), jnp.int32),
            pltpu.VMEM((2, _GCHUNK, _CHUNK), jnp.int32),
            pltpu.VMEM((_CHUNK, _D), jnp.float32),
            pltpu.VMEM((_CHUNK, _D), jnp.float32),
            pltpu.VMEM((_CHUNK, _D), jnp.float32),
            pltpu.VMEM((_CHUNK, _D), jnp.float32),
            pltpu.VMEM_SHARED((_ACC_ROWS, _D), jnp.float32),
            pltpu.SemaphoreType.DMA,
            pltpu.SemaphoreType.DMA,
            pltpu.SemaphoreType.DMA,
            pltpu.SemaphoreType.DMA,
            pltpu.SemaphoreType.DMA,
            pltpu.SemaphoreType.DMA,
            pltpu.SemaphoreType.DMA,
            pltpu.SemaphoreType.DMA,
            pltpu.SemaphoreType.DMA,
        ],
    )
    partials = sc_call(features, src_p, dst_p, zeros)

    out = pl.pallas_call(
        _addmm_body,
        grid=(_N_NODES // _MM_BLOCK,),
        in_specs=[
            pl.BlockSpec((_MM_BLOCK, _D), lambda i: (i, 0)),
            pl.BlockSpec((_MM_BLOCK, _D), lambda i: (i, 0)),
            pl.BlockSpec((_D, _D), lambda i: (0, 0)),
        ],
        out_specs=pl.BlockSpec((_MM_BLOCK, _D), lambda i: (i, 0)),
        out_shape=jax.ShapeDtypeStruct((_N_NODES, _D), jnp.float32),
    )(partials[0], partials[1], W)
    return out
